# baseline (speedup 1.0000x reference)
import functools
import math

import jax
import jax.numpy as jnp
from jax import lax
from jax.experimental import pallas as pl
from jax.experimental.pallas import tpu as pltpu
from jax.experimental.pallas import tpu_sc as plsc

_TB = 2048
_SUB = 8192
_MP = 368


def _tc_body(w_ref, p_ref, a_ref, d2_ref):
    w = w_ref[...]
    p = p_ref[...]
    res = jnp.dot(w, p, preferred_element_type=jnp.float32)
    xv = res[0:24]
    yv = res[24:48]
    zv = res[48:72]
    a_ref[...] = jnp.arctan2(yv, xv)
    d2_ref[...] = xv * xv + yv * yv + zv * zv + 1.0


def _tc_stage(w_all, pts4):
    n = pts4.shape[1]
    return pl.pallas_call(
        _tc_body,
        grid=(n // _TB,),
        in_specs=[
            pl.BlockSpec((72, 4), lambda i: (0, 0)),
            pl.BlockSpec((4, _TB), lambda i: (0, i)),
        ],
        out_specs=[
            pl.BlockSpec((24, _TB), lambda i: (0, i)),
            pl.BlockSpec((24, _TB), lambda i: (0, i)),
        ],
        out_shape=[
            jax.ShapeDtypeStruct((24, n), jnp.float32),
            jax.ShapeDtypeStruct((24, n), jnp.float32),
        ],
    )(w_all, pts4)


def _sc_stage(a24, d224, al_tab, dl_tab, sl_tab, g0s, n, t, m):
    nc, ns, nl = 2, 16, 16
    try:
        info = plsc.get_sparse_core_info()
        nc, ns, nl = info.num_cores, info.num_subcores, info.num_lanes
    except Exception:
        pass
    nw = nc * ns
    chunk = n // nw
    nsub = chunk // _SUB
    tm = t * _MP
    invd = jnp.float32(1.0 / (2.0 * math.pi / (m - 2)))
    mesh = plsc.VectorSubcoreMesh(core_axis_name="c", subcore_axis_name="s")

    @functools.partial(
        pl.kernel,
        out_type=jax.ShapeDtypeStruct((n,), jnp.int32),
        mesh=mesh,
        scratch_types=[
            pltpu.VMEM_SHARED((t * _MP,), jnp.float32),
            pltpu.VMEM_SHARED((t * _MP,), jnp.float32),
            pltpu.VMEM_SHARED((t * _MP,), jnp.float32),
            [pltpu.VMEM((_SUB,), jnp.float32)] * 2,
            [pltpu.VMEM((_SUB,), jnp.float32)] * 2,
            [pltpu.VMEM((_SUB,), jnp.int32)] * 2,
            [pltpu.VMEM((_SUB,), jnp.float32)] * 2,
            [pltpu.VMEM((_SUB,), jnp.float32)] * 2,
            [pltpu.VMEM((_SUB,), jnp.float32)] * 2,
            pltpu.VMEM((_SUB,), jnp.int32),
            pltpu.VMEM((16,), jnp.float32),
            [pltpu.SemaphoreType.DMA] * 2,
        ],
    )
    def sc_kernel(a_hbm, d2_hbm, al_hbm, dl_hbm, sl_hbm, g0_hbm, out_hbm,
                  al_s, dl_s, sl_s, a_v, d2_v, idx_v, alg_v, dlg_v, slg_v,
                  out_v, g0_v, sem):
        cid = lax.axis_index("c")
        sid = lax.axis_index("s")
        wid = sid * nc + cid
        base = wid * chunk

        @pl.when(sid == 0)
        def _():
            pltpu.sync_copy(al_hbm, al_s)
            pltpu.sync_copy(dl_hbm, dl_s)
            pltpu.sync_copy(sl_hbm, sl_s)
        plsc.subcore_barrier()

        pltpu.sync_copy(g0_hbm, g0_v)
        g0 = g0_v[pl.ds(0, nl)]
        nv = _SUB // nl

        def load_rows(ts, sbase, par):
            pltpu.sync_copy(a_hbm.at[pl.ds(ts * n + sbase, _SUB)], a_v[par])
            pltpu.sync_copy(d2_hbm.at[pl.ds(ts * n + sbase, _SUB)], d2_v[par])

        def calc_idx(ts, par):
            @plsc.parallel_loop(0, _SUB, step=nl, unroll=4)
            def idx_body(i):
                sl = pl.ds(i, nl)
                av = a_v[par][sl]
                fidx = (av - g0) * invd
                idx = jnp.clip(fidx.astype(jnp.int32) + 1, 1, m - 1)
                idx_v[par][sl] = idx + ts * _MP

        def fire_gathers(par):
            pltpu.async_copy(al_s.at[idx_v[par]], alg_v[par], sem[par])
            pltpu.async_copy(dl_s.at[idx_v[par]], dlg_v[par], sem[par])
            pltpu.async_copy(sl_s.at[idx_v[par]], slg_v[par], sem[par])

        def wait_gathers(par):
            pltpu.make_async_copy(
                a_hbm.at[pl.ds(0, _SUB)], alg_v[par], sem[par]).wait()
            pltpu.make_async_copy(
                a_hbm.at[pl.ds(0, _SUB)], dlg_v[par], sem[par]).wait()
            pltpu.make_async_copy(
                a_hbm.at[pl.ds(0, _SUB)], slg_v[par], sem[par]).wait()

        for sub in range(nsub):
            sbase = base + sub * _SUB

            @plsc.parallel_loop(0, _SUB, step=nl, unroll=4)
            def zero_body(i):
                out_v[pl.ds(i, nl)] = jnp.zeros((nl,), jnp.int32)

            load_rows(0, sbase, 0)
            calc_idx(0, 0)
            fire_gathers(0)
            load_rows(1, sbase, 1)

            def pair_body(i, c):
                for par in (0, 1):
                    ts = 2 * i + par
                    wait_gathers(par)
                    w = jnp.where(ts == 0, jnp.int32(33), jnp.int32(1))
                    nxt = 1 - par

                    calc_idx(ts + 1, nxt)

                    @pl.when(ts < t - 1)
                    def _():
                        fire_gathers(nxt)

                    @plsc.parallel_loop(0, _SUB, step=nl, unroll=4)
                    def occ_body(j):
                        sl = pl.ds(j, nl)
                        av = a_v[par][sl]
                        dv = d2_v[par][sl]
                        surf = dlg_v[par][sl] + slg_v[par][sl] * (
                            av - alg_v[par][sl])
                        t1 = lax.bitcast_convert_type(
                            surf * surf - dv, jnp.int32)
                        t2 = lax.bitcast_convert_type(
                            dv - jnp.float32(9.0), jnp.int32)
                        occi = lax.shift_right_logical(t1 | t2, 31)
                        out_v[sl] = out_v[sl] + occi * w

                    @pl.when(ts < t - 2)
                    def _():
                        load_rows(ts + 2, sbase, par)
                return c

            lax.fori_loop(0, t // 2, pair_body, 0)

            pltpu.sync_copy(out_v, out_hbm.at[pl.ds(sbase, _SUB)])

    return sc_kernel(a24.reshape(-1), d224.reshape(-1), al_tab, dl_tab, sl_tab, g0s)


def kernel(pts, lidar_polar, velo_poses):
    n = pts.shape[0]
    t = lidar_polar.shape[0]
    m = lidar_polar.shape[1]
    w2v = jnp.linalg.inv(velo_poses)
    w_all = jnp.zeros((72, 4), jnp.float32)
    w_all = w_all.at[0:t].set(w2v[:, 0, :])
    w_all = w_all.at[24:24 + t].set(w2v[:, 1, :])
    w_all = w_all.at[48:48 + t].set(w2v[:, 2, :])
    pts4 = jnp.concatenate(
        [pts.T, jnp.ones((1, n), dtype=pts.dtype)], axis=0)

    half = n // 2
    a24a, d224a = _tc_stage(w_all, pts4[:, :half])
    a24b, d224b = _tc_stage(w_all, pts4[:, half:])

    ang = lidar_polar[0, :, 0]
    dist = lidar_polar[:, :, 1]
    al_row = jnp.concatenate([ang[:1], ang[:-1]])
    al_tab = jnp.broadcast_to(al_row, (t, m))
    dl_tab = jnp.concatenate([dist[:, :1], dist[:, :-1]], axis=1)
    slope = (dist[:, 1:] - dist[:, :-1]) / (ang[1:] - ang[:-1])
    sl_tab = jnp.concatenate([jnp.zeros((t, 1), jnp.float32), slope], axis=1)

    def flat(x):
        return jnp.pad(x, ((0, 0), (0, _MP - m))).reshape(-1)

    g0s = jnp.full((16,), ang[0], jnp.float32)
    enc0 = _sc_stage(a24a, d224a, flat(al_tab), flat(dl_tab), flat(sl_tab),
                     g0s, half, t, m)
    enc1 = _sc_stage(a24b, d224b, flat(al_tab), flat(dl_tab), flat(sl_tab),
                     g0s, half, t, m)
    enc = jnp.concatenate([enc0, enc1])

    count = enc & 31
    occ0 = enc >> 5
    is_occupied = (1.0 + count.astype(jnp.float32)) / t > (t - 2) / t
    is_visible = occ0 == 0
    return (is_occupied, is_visible)


# speedup vs baseline: 1.1433x; 1.1433x over previous
import functools
import math

import jax
import jax.numpy as jnp
from jax import lax
from jax.experimental import pallas as pl
from jax.experimental.pallas import tpu as pltpu
from jax.experimental.pallas import tpu_sc as plsc

_TB = 2048
_SUB = 8192
_MP = 368


def _tc_body(w_ref, p_ref, a_ref, d2_ref):
    w = w_ref[...]
    p = p_ref[...]
    res = jnp.dot(w, p, preferred_element_type=jnp.float32)
    xv = res[0:24]
    yv = res[24:48]
    zv = res[48:72]
    a_ref[...] = jnp.arctan2(yv, xv)
    d2_ref[...] = xv * xv + yv * yv + zv * zv + 1.0


def _tc_stage(w_all, pts4):
    n = pts4.shape[1]
    return pl.pallas_call(
        _tc_body,
        grid=(n // _TB,),
        in_specs=[
            pl.BlockSpec((72, 4), lambda i: (0, 0)),
            pl.BlockSpec((4, _TB), lambda i: (0, i)),
        ],
        out_specs=[
            pl.BlockSpec((24, _TB), lambda i: (0, i)),
            pl.BlockSpec((24, _TB), lambda i: (0, i)),
        ],
        out_shape=[
            jax.ShapeDtypeStruct((24, n), jnp.float32),
            jax.ShapeDtypeStruct((24, n), jnp.float32),
        ],
    )(w_all, pts4)


def _sc_stage(a24, d224, al_tab, dl_tab, sl_tab, g0s, n, t, m):
    nc, ns, nl = 2, 16, 16
    try:
        info = plsc.get_sparse_core_info()
        nc, ns, nl = info.num_cores, info.num_subcores, info.num_lanes
    except Exception:
        pass
    nw = nc * ns
    chunk = n // nw
    nsub = chunk // _SUB
    tm = t * _MP
    invd = jnp.float32(1.0 / (2.0 * math.pi / (m - 2)))
    mesh = plsc.VectorSubcoreMesh(core_axis_name="c", subcore_axis_name="s")

    @functools.partial(
        pl.kernel,
        out_type=jax.ShapeDtypeStruct((n,), jnp.int32),
        mesh=mesh,
        scratch_types=[
            pltpu.VMEM_SHARED((t * _MP,), jnp.float32),
            pltpu.VMEM_SHARED((t * _MP,), jnp.float32),
            pltpu.VMEM_SHARED((t * _MP,), jnp.float32),
            [pltpu.VMEM((_SUB,), jnp.float32)] * 2,
            [pltpu.VMEM((_SUB,), jnp.float32)] * 2,
            [pltpu.VMEM((_SUB,), jnp.int32)] * 2,
            [pltpu.VMEM((_SUB,), jnp.float32)] * 2,
            [pltpu.VMEM((_SUB,), jnp.float32)] * 2,
            [pltpu.VMEM((_SUB,), jnp.float32)] * 2,
            pltpu.VMEM((_SUB,), jnp.int32),
            pltpu.VMEM((16,), jnp.float32),
            [pltpu.SemaphoreType.DMA] * 2,
        ],
    )
    def sc_kernel(a_hbm, d2_hbm, al_hbm, dl_hbm, sl_hbm, g0_hbm, out_hbm,
                  al_s, dl_s, sl_s, a_v, d2_v, idx_v, alg_v, dlg_v, slg_v,
                  out_v, g0_v, sem):
        cid = lax.axis_index("c")
        sid = lax.axis_index("s")
        wid = sid * nc + cid
        base = wid * chunk

        @pl.when(sid == 0)
        def _():
            pltpu.sync_copy(al_hbm, al_s)
            pltpu.sync_copy(dl_hbm, dl_s)
            pltpu.sync_copy(sl_hbm, sl_s)
        plsc.subcore_barrier()

        pltpu.sync_copy(g0_hbm, g0_v)
        g0 = g0_v[pl.ds(0, nl)]
        nv = _SUB // nl

        def load_rows(ts, sbase, par):
            pltpu.sync_copy(a_hbm.at[pl.ds(ts * n + sbase, _SUB)], a_v[par])
            pltpu.sync_copy(d2_hbm.at[pl.ds(ts * n + sbase, _SUB)], d2_v[par])

        def calc_idx(ts, par):
            @plsc.parallel_loop(0, _SUB, step=nl, unroll=4)
            def idx_body(i):
                sl = pl.ds(i, nl)
                av = a_v[par][sl]
                fidx = (av - g0) * invd
                idx = jnp.clip(fidx.astype(jnp.int32) + 1, 1, m - 1)
                idx_v[par][sl] = idx + ts * _MP

        def fire_gathers(par):
            pltpu.async_copy(al_s.at[idx_v[par]], alg_v[par], sem[par])
            pltpu.async_copy(dl_s.at[idx_v[par]], dlg_v[par], sem[par])
            pltpu.async_copy(sl_s.at[idx_v[par]], slg_v[par], sem[par])

        def wait_gathers(par):
            pltpu.make_async_copy(
                a_hbm.at[pl.ds(0, _SUB)], alg_v[par], sem[par]).wait()
            pltpu.make_async_copy(
                a_hbm.at[pl.ds(0, _SUB)], dlg_v[par], sem[par]).wait()
            pltpu.make_async_copy(
                a_hbm.at[pl.ds(0, _SUB)], slg_v[par], sem[par]).wait()

        for sub in range(nsub):
            sbase = base + sub * _SUB

            @plsc.parallel_loop(0, _SUB, step=nl, unroll=4)
            def zero_body(i):
                out_v[pl.ds(i, nl)] = jnp.zeros((nl,), jnp.int32)

            load_rows(0, sbase, 0)
            calc_idx(0, 0)
            fire_gathers(0)
            load_rows(1, sbase, 1)

            def pair_body(i, c):
                for par in (0, 1):
                    ts = 2 * i + par
                    wait_gathers(par)
                    w = jnp.where(ts == 0, jnp.int32(33), jnp.int32(1))
                    nxt = 1 - par

                    calc_idx(ts + 1, nxt)

                    @pl.when(ts < t - 1)
                    def _():
                        fire_gathers(nxt)

                    @plsc.parallel_loop(0, _SUB, step=nl, unroll=4)
                    def occ_body(j):
                        sl = pl.ds(j, nl)
                        av = a_v[par][sl]
                        dv = d2_v[par][sl]
                        surf = dlg_v[par][sl] + slg_v[par][sl] * (
                            av - alg_v[par][sl])
                        t1 = lax.bitcast_convert_type(
                            surf * surf - dv, jnp.int32)
                        t2 = lax.bitcast_convert_type(
                            dv - jnp.float32(9.0), jnp.int32)
                        occi = lax.shift_right_logical(t1 | t2, 31)
                        out_v[sl] = out_v[sl] + occi * w

                    @pl.when(ts < t - 2)
                    def _():
                        load_rows(ts + 2, sbase, par)
                return c

            lax.fori_loop(0, t // 2, pair_body, 0)

            pltpu.sync_copy(out_v, out_hbm.at[pl.ds(sbase, _SUB)])

    return sc_kernel(a24.reshape(-1), d224.reshape(-1), al_tab, dl_tab, sl_tab, g0s)


def kernel(pts, lidar_polar, velo_poses):
    n = pts.shape[0]
    t = lidar_polar.shape[0]
    m = lidar_polar.shape[1]
    w2v = jnp.linalg.inv(velo_poses)
    w_all = jnp.zeros((72, 4), jnp.float32)
    w_all = w_all.at[0:t].set(w2v[:, 0, :])
    w_all = w_all.at[24:24 + t].set(w2v[:, 1, :])
    w_all = w_all.at[48:48 + t].set(w2v[:, 2, :])
    pts4 = jnp.concatenate(
        [pts.T, jnp.ones((1, n), dtype=pts.dtype)], axis=0)

    a24, d224 = _tc_stage(w_all, pts4)

    ang = lidar_polar[0, :, 0]
    dist = lidar_polar[:, :, 1]
    al_row = jnp.concatenate([ang[:1], ang[:-1]])
    al_tab = jnp.broadcast_to(al_row, (t, m))
    dl_tab = jnp.concatenate([dist[:, :1], dist[:, :-1]], axis=1)
    slope = (dist[:, 1:] - dist[:, :-1]) / (ang[1:] - ang[:-1])
    sl_tab = jnp.concatenate([jnp.zeros((t, 1), jnp.float32), slope], axis=1)

    def flat(x):
        return jnp.pad(x, ((0, 0), (0, _MP - m))).reshape(-1)

    g0s = jnp.full((16,), ang[0], jnp.float32)
    enc = _sc_stage(a24, d224, flat(al_tab), flat(dl_tab), flat(sl_tab),
                    g0s, n, t, m)

    count = enc & 31
    occ0 = enc >> 5
    is_occupied = (1.0 + count.astype(jnp.float32)) / t > (t - 2) / t
    is_visible = occ0 == 0
    return (is_occupied, is_visible)
